# baseline (speedup 1.0000x reference)
import functools

import jax
import jax.numpy as jnp
from jax import lax
from jax.experimental import pallas as pl
from jax.experimental.pallas import tpu as pltpu
from jax.experimental.pallas import tpu_sc as plsc

N = 10000
IN_CH = 128
HID = 8
HEADS = 8
F1 = HEADS * HID
OUT_CH = 32

NC, NS = 2, 16
NTILES = NC * NS
CHUNK = 128
NPAD = 10240
DUMMY = N
ROWS_PER_SUB = NPAD // NS

_EPS = 1e-16


def _ceil_to(a, b):
    return (a + b - 1) // b * b


def _make_sc_pass(epad, feat):
    edges_per_tile = epad // NTILES
    steps = edges_per_tile // CHUNK
    assert steps % 2 == 0
    nvec = feat // 16
    w = 2 * feat
    mesh = plsc.VectorSubcoreMesh(core_axis_name="c", subcore_axis_name="s")

    @functools.partial(
        pl.kernel,
        out_type=jax.ShapeDtypeStruct((NC, NPAD, w), jnp.float32),
        mesh=mesh,
        compiler_params=pltpu.CompilerParams(use_tc_tiling_on_sc=False),
        scratch_types=[
            pltpu.VMEM((CHUNK,), jnp.int32),
            pltpu.VMEM((CHUNK,), jnp.int32),
            pltpu.VMEM((CHUNK,), jnp.int32),
            pltpu.VMEM((CHUNK,), jnp.int32),
            pltpu.VMEM((CHUNK,), jnp.int32),
            pltpu.VMEM((CHUNK, w), jnp.float32),
            pltpu.VMEM((CHUNK, feat), jnp.float32),
            pltpu.VMEM((CHUNK, w), jnp.float32),
            pltpu.SemaphoreType.DMA,
            pltpu.SemaphoreType.DMA,
            pltpu.SemaphoreType.DMA,
            pltpu.VMEM_SHARED((NPAD, w), jnp.float32),
        ],
    )
    def sc_pass(src_hbm, dst_hbm, srct_hbm, adx_hbm, zeros_hbm, out_hbm,
                is0, is1, id0, id1, sdst, srcrows, adxrows, contrib,
                ix0, ix1, gsem, acc):
        isrc = (is0, is1)
        idst = (id0, id1)
        isem = (ix0, ix1)
        cid = lax.axis_index("c")
        sid = lax.axis_index("s")
        wid = cid * NS + sid

        pltpu.sync_copy(zeros_hbm.at[pl.ds(sid * ROWS_PER_SUB, ROWS_PER_SUB)],
                        acc.at[pl.ds(sid * ROWS_PER_SUB, ROWS_PER_SUB)])
        plsc.subcore_barrier()

        base0 = wid * edges_per_tile

        def issue_idx(i, b):
            pltpu.async_copy(src_hbm.at[pl.ds(base0 + i * CHUNK, CHUNK)],
                             isrc[b], isem[b])
            pltpu.async_copy(dst_hbm.at[pl.ds(base0 + i * CHUNK, CHUNK)],
                             idst[b], isem[b])

        def wait_idx(i, b):
            pltpu.make_async_copy(src_hbm.at[pl.ds(base0 + i * CHUNK, CHUNK)],
                                  isrc[b], isem[b]).wait()
            pltpu.make_async_copy(dst_hbm.at[pl.ds(base0 + i * CHUNK, CHUNK)],
                                  idst[b], isem[b]).wait()

        issue_idx(0, 0)
        issue_idx(1, 1)

        @pl.loop(0, steps // 2)
        def _(p):
            for b in range(2):
                i = 2 * p + b

                wait_idx(i, b)
                pltpu.async_copy(srct_hbm.at[isrc[b]], srcrows, gsem)
                pltpu.async_copy(adx_hbm.at[idst[b]], adxrows, gsem)

                for q in range(CHUNK // 16):
                    qs = pl.ds(q * 16, 16)
                    sdst[qs] = idst[b][qs]

                pltpu.make_async_copy(srct_hbm.at[isrc[b]], srcrows,
                                      gsem).wait()
                pltpu.make_async_copy(adx_hbm.at[idst[b]], adxrows,
                                      gsem).wait()

                @pl.when(i + 2 < steps)
                def _():
                    issue_idx(i + 2, b)

                @pl.loop(0, CHUNK)
                def _(e):
                    for v in range(nvec):
                        sl = pl.ds(16 * v, 16)
                        sh = pl.ds(feat + 16 * v, 16)
                        a = srcrows[e, sh] + adxrows[e, sl]
                        ev = jnp.exp(jnp.maximum(a, 0.2 * a))
                        contrib[e, sl] = srcrows[e, sl] * ev
                        contrib[e, sh] = ev

                pltpu.sync_copy(contrib, acc.at[sdst], add=True)

        plsc.subcore_barrier()
        pltpu.sync_copy(acc.at[pl.ds(sid * ROWS_PER_SUB, ROWS_PER_SUB)],
                        out_hbm.at[cid, pl.ds(sid * ROWS_PER_SUB, ROWS_PER_SUB)])

    return sc_pass


_BLK = 256
_HI = lax.Precision.HIGHEST


def _dot(a, b):
    return jnp.dot(a, b, precision=_HI, preferred_element_type=jnp.float32)


def _tc1(xp, W1, AexpS, AexpD):
    def body(x_ref, w_ref, s_ref, d_ref, t_ref, adx_ref):
        h = _dot(x_ref[...], w_ref[...])
        t_ref[:, :F1] = h
        t_ref[:, F1:] = _dot(h, s_ref[...])
        adx_ref[...] = _dot(h, d_ref[...])

    grid = (NPAD // _BLK,)
    return pl.pallas_call(
        body,
        grid=grid,
        in_specs=[
            pl.BlockSpec((_BLK, IN_CH), lambda i: (i, 0)),
            pl.BlockSpec((IN_CH, F1), lambda i: (0, 0)),
            pl.BlockSpec((F1, F1), lambda i: (0, 0)),
            pl.BlockSpec((F1, F1), lambda i: (0, 0)),
        ],
        out_specs=[
            pl.BlockSpec((_BLK, 2 * F1), lambda i: (i, 0)),
            pl.BlockSpec((_BLK, F1), lambda i: (i, 0)),
        ],
        out_shape=[
            jax.ShapeDtypeStruct((NPAD, 2 * F1), jnp.float32),
            jax.ShapeDtypeStruct((NPAD, F1), jnp.float32),
        ],
    )(xp, W1, AexpS, AexpD)


def _tc2(acc1, b1, W2, att_src2_t, att_dst2_t):
    def body(a_ref, b_ref, w_ref, s_ref, d_ref, t_ref, adx_ref):
        s = a_ref[0] + a_ref[1]
        msg = s[:, :F1]
        den = s[:, F1:]
        h1o = jnp.maximum(msg / (den + _EPS) + b_ref[...], 0.0)
        h2 = _dot(h1o, w_ref[...])
        a_s = _dot(h2, s_ref[...])
        a_d = _dot(h2, d_ref[...])
        t_ref[:, :OUT_CH] = h2
        t_ref[:, OUT_CH:] = jnp.broadcast_to(a_s, (_BLK, OUT_CH))
        adx_ref[...] = jnp.broadcast_to(a_d, (_BLK, OUT_CH))

    grid = (NPAD // _BLK,)
    return pl.pallas_call(
        body,
        grid=grid,
        in_specs=[
            pl.BlockSpec((NC, _BLK, 2 * F1), lambda i: (0, i, 0)),
            pl.BlockSpec((1, F1), lambda i: (0, 0)),
            pl.BlockSpec((F1, OUT_CH), lambda i: (0, 0)),
            pl.BlockSpec((OUT_CH, 1), lambda i: (0, 0)),
            pl.BlockSpec((OUT_CH, 1), lambda i: (0, 0)),
        ],
        out_specs=[
            pl.BlockSpec((_BLK, 2 * OUT_CH), lambda i: (i, 0)),
            pl.BlockSpec((_BLK, OUT_CH), lambda i: (i, 0)),
        ],
        out_shape=[
            jax.ShapeDtypeStruct((NPAD, 2 * OUT_CH), jnp.float32),
            jax.ShapeDtypeStruct((NPAD, OUT_CH), jnp.float32),
        ],
    )(acc1, b1, W2, att_src2_t, att_dst2_t)


def _tc3(acc2, b2):
    def body(a_ref, b_ref, o_ref):
        s = a_ref[0] + a_ref[1]
        msg = s[:, :OUT_CH]
        den = s[:, OUT_CH:]
        logits = msg / (den + _EPS) + b_ref[...]
        m = jnp.max(logits, axis=1, keepdims=True)
        lse = jnp.log(jnp.sum(jnp.exp(logits - m), axis=1, keepdims=True)) + m
        o_ref[...] = logits - lse

    grid = (NPAD // _BLK,)
    return pl.pallas_call(
        body,
        grid=grid,
        in_specs=[
            pl.BlockSpec((NC, _BLK, 2 * OUT_CH), lambda i: (0, i, 0)),
            pl.BlockSpec((1, OUT_CH), lambda i: (0, 0)),
        ],
        out_specs=pl.BlockSpec((_BLK, OUT_CH), lambda i: (i, 0)),
        out_shape=jax.ShapeDtypeStruct((NPAD, OUT_CH), jnp.float32),
    )(acc2, b2)


def kernel(x, edge_index, W1, att_src1, att_dst1, b1, W2, att_src2, att_dst2, b2):
    e_raw = edge_index.shape[1]
    etot = e_raw + N
    epad = _ceil_to(etot, NTILES * CHUNK * 2)

    loop_idx = jnp.arange(N, dtype=jnp.int32)
    padv = jnp.full((epad - etot,), DUMMY, dtype=jnp.int32)
    src = jnp.concatenate([edge_index[0].astype(jnp.int32), loop_idx, padv])
    dst = jnp.concatenate([edge_index[1].astype(jnp.int32), loop_idx, padv])

    eye = jnp.eye(HEADS, dtype=jnp.float32)
    AexpS = (eye[:, None, :, None] * att_src1[:, :, None, None]
             * jnp.ones((1, 1, 1, HID), jnp.float32)).reshape(F1, F1)
    AexpD = (eye[:, None, :, None] * att_dst1[:, :, None, None]
             * jnp.ones((1, 1, 1, HID), jnp.float32)).reshape(F1, F1)

    xp = jnp.zeros((NPAD, IN_CH), jnp.float32).at[:N].set(x)

    srct1, adx1 = _tc1(xp, W1, AexpS, AexpD)

    sc1 = _make_sc_pass(epad, F1)
    acc1 = sc1(src, dst, srct1, adx1, jnp.zeros((NPAD, 2 * F1), jnp.float32))

    srct2, adx2 = _tc2(acc1, b1.reshape(1, F1), W2,
                       att_src2.reshape(OUT_CH, 1), att_dst2.reshape(OUT_CH, 1))

    sc2 = _make_sc_pass(epad, OUT_CH)
    acc2 = sc2(src, dst, srct2, adx2,
               jnp.zeros((NPAD, 2 * OUT_CH), jnp.float32))

    out = _tc3(acc2, b2.reshape(1, OUT_CH))
    return out[:N]


# speedup vs baseline: 1.0323x; 1.0323x over previous
import functools

import jax
import jax.numpy as jnp
from jax import lax
from jax.experimental import pallas as pl
from jax.experimental.pallas import tpu as pltpu
from jax.experimental.pallas import tpu_sc as plsc

N = 10000
IN_CH = 128
HID = 8
HEADS = 8
F1 = HEADS * HID
OUT_CH = 32

NC, NS = 2, 16
NTILES = NC * NS
CHUNK = 128
NPAD = 10240
DUMMY = N
ROWS_PER_SUB = NPAD // NS

_EPS = 1e-16


def _ceil_to(a, b):
    return (a + b - 1) // b * b


def _make_sc_pass(epad, feat):
    edges_per_tile = epad // NTILES
    steps = edges_per_tile // CHUNK
    nvec = feat // 16
    w = 2 * feat
    mesh = plsc.VectorSubcoreMesh(core_axis_name="c", subcore_axis_name="s")

    @functools.partial(
        pl.kernel,
        out_type=jax.ShapeDtypeStruct((NC, NPAD, w), jnp.float32),
        mesh=mesh,
        compiler_params=pltpu.CompilerParams(use_tc_tiling_on_sc=False),
        scratch_types=[
            pltpu.VMEM((CHUNK,), jnp.int32),
            pltpu.VMEM((CHUNK,), jnp.int32),
            pltpu.VMEM((CHUNK, w), jnp.float32),
            pltpu.VMEM((CHUNK, feat), jnp.float32),
            pltpu.VMEM((CHUNK, w), jnp.float32),
            pltpu.VMEM_SHARED((NPAD, w), jnp.float32),
        ],
    )
    def sc_pass(src_hbm, dst_hbm, srct_hbm, adx_hbm, zeros_hbm, out_hbm,
                isrc, idst, srcrows, adxrows, contrib, acc):
        cid = lax.axis_index("c")
        sid = lax.axis_index("s")
        wid = cid * NS + sid

        pltpu.sync_copy(zeros_hbm.at[pl.ds(sid * ROWS_PER_SUB, ROWS_PER_SUB)],
                        acc.at[pl.ds(sid * ROWS_PER_SUB, ROWS_PER_SUB)])
        plsc.subcore_barrier()

        base0 = wid * edges_per_tile

        @pl.loop(0, steps)
        def _(i):
            base = base0 + i * CHUNK
            pltpu.sync_copy(src_hbm.at[pl.ds(base, CHUNK)], isrc)
            pltpu.sync_copy(dst_hbm.at[pl.ds(base, CHUNK)], idst)
            pltpu.sync_copy(srct_hbm.at[isrc], srcrows)
            pltpu.sync_copy(adx_hbm.at[idst], adxrows)

            @pl.loop(0, CHUNK)
            def _(e):
                for v in range(nvec):
                    sl = pl.ds(16 * v, 16)
                    sh = pl.ds(feat + 16 * v, 16)
                    a = srcrows[e, sh] + adxrows[e, sl]
                    ev = jnp.exp(jnp.maximum(a, 0.2 * a))
                    contrib[e, sl] = srcrows[e, sl] * ev
                    contrib[e, sh] = ev

            pltpu.sync_copy(contrib, acc.at[idst], add=True)

        plsc.subcore_barrier()
        pltpu.sync_copy(acc.at[pl.ds(sid * ROWS_PER_SUB, ROWS_PER_SUB)],
                        out_hbm.at[cid, pl.ds(sid * ROWS_PER_SUB, ROWS_PER_SUB)])

    return sc_pass


_BLK = 256
_HI = lax.Precision.HIGHEST


def _dot(a, b):
    return jnp.dot(a, b, precision=_HI, preferred_element_type=jnp.float32)


def _tc1(xp, W1, AexpS, AexpD):
    def body(x_ref, w_ref, s_ref, d_ref, t_ref, adx_ref):
        h = _dot(x_ref[...], w_ref[...])
        t_ref[:, :F1] = h
        t_ref[:, F1:] = _dot(h, s_ref[...])
        adx_ref[...] = _dot(h, d_ref[...])

    grid = (NPAD // _BLK,)
    return pl.pallas_call(
        body,
        grid=grid,
        in_specs=[
            pl.BlockSpec((_BLK, IN_CH), lambda i: (i, 0)),
            pl.BlockSpec((IN_CH, F1), lambda i: (0, 0)),
            pl.BlockSpec((F1, F1), lambda i: (0, 0)),
            pl.BlockSpec((F1, F1), lambda i: (0, 0)),
        ],
        out_specs=[
            pl.BlockSpec((_BLK, 2 * F1), lambda i: (i, 0)),
            pl.BlockSpec((_BLK, F1), lambda i: (i, 0)),
        ],
        out_shape=[
            jax.ShapeDtypeStruct((NPAD, 2 * F1), jnp.float32),
            jax.ShapeDtypeStruct((NPAD, F1), jnp.float32),
        ],
    )(xp, W1, AexpS, AexpD)


def _tc2(acc1, b1, W2, att_src2_t, att_dst2_t):
    def body(a_ref, b_ref, w_ref, s_ref, d_ref, t_ref, adx_ref):
        s = a_ref[0] + a_ref[1]
        msg = s[:, :F1]
        den = s[:, F1:]
        h1o = jnp.maximum(msg / (den + _EPS) + b_ref[...], 0.0)
        h2 = _dot(h1o, w_ref[...])
        a_s = _dot(h2, s_ref[...])
        a_d = _dot(h2, d_ref[...])
        t_ref[:, :OUT_CH] = h2
        t_ref[:, OUT_CH:] = jnp.broadcast_to(a_s, (_BLK, OUT_CH))
        adx_ref[...] = jnp.broadcast_to(a_d, (_BLK, OUT_CH))

    grid = (NPAD // _BLK,)
    return pl.pallas_call(
        body,
        grid=grid,
        in_specs=[
            pl.BlockSpec((NC, _BLK, 2 * F1), lambda i: (0, i, 0)),
            pl.BlockSpec((1, F1), lambda i: (0, 0)),
            pl.BlockSpec((F1, OUT_CH), lambda i: (0, 0)),
            pl.BlockSpec((OUT_CH, 1), lambda i: (0, 0)),
            pl.BlockSpec((OUT_CH, 1), lambda i: (0, 0)),
        ],
        out_specs=[
            pl.BlockSpec((_BLK, 2 * OUT_CH), lambda i: (i, 0)),
            pl.BlockSpec((_BLK, OUT_CH), lambda i: (i, 0)),
        ],
        out_shape=[
            jax.ShapeDtypeStruct((NPAD, 2 * OUT_CH), jnp.float32),
            jax.ShapeDtypeStruct((NPAD, OUT_CH), jnp.float32),
        ],
    )(acc1, b1, W2, att_src2_t, att_dst2_t)


def _tc3(acc2, b2):
    def body(a_ref, b_ref, o_ref):
        s = a_ref[0] + a_ref[1]
        msg = s[:, :OUT_CH]
        den = s[:, OUT_CH:]
        logits = msg / (den + _EPS) + b_ref[...]
        m = jnp.max(logits, axis=1, keepdims=True)
        lse = jnp.log(jnp.sum(jnp.exp(logits - m), axis=1, keepdims=True)) + m
        o_ref[...] = logits - lse

    grid = (NPAD // _BLK,)
    return pl.pallas_call(
        body,
        grid=grid,
        in_specs=[
            pl.BlockSpec((NC, _BLK, 2 * OUT_CH), lambda i: (0, i, 0)),
            pl.BlockSpec((1, OUT_CH), lambda i: (0, 0)),
        ],
        out_specs=pl.BlockSpec((_BLK, OUT_CH), lambda i: (i, 0)),
        out_shape=jax.ShapeDtypeStruct((NPAD, OUT_CH), jnp.float32),
    )(acc2, b2)


def kernel(x, edge_index, W1, att_src1, att_dst1, b1, W2, att_src2, att_dst2, b2):
    e_raw = edge_index.shape[1]
    etot = e_raw + N
    epad = _ceil_to(etot, NTILES * CHUNK)

    loop_idx = jnp.arange(N, dtype=jnp.int32)
    padv = jnp.full((epad - etot,), DUMMY, dtype=jnp.int32)
    src = jnp.concatenate([edge_index[0].astype(jnp.int32), loop_idx, padv])
    dst = jnp.concatenate([edge_index[1].astype(jnp.int32), loop_idx, padv])

    eye = jnp.eye(HEADS, dtype=jnp.float32)
    AexpS = (eye[:, None, :, None] * att_src1[:, :, None, None]
             * jnp.ones((1, 1, 1, HID), jnp.float32)).reshape(F1, F1)
    AexpD = (eye[:, None, :, None] * att_dst1[:, :, None, None]
             * jnp.ones((1, 1, 1, HID), jnp.float32)).reshape(F1, F1)

    xp = jnp.zeros((NPAD, IN_CH), jnp.float32).at[:N].set(x)

    srct1, adx1 = _tc1(xp, W1, AexpS, AexpD)

    sc1 = _make_sc_pass(epad, F1)
    acc1 = sc1(src, dst, srct1, adx1, jnp.zeros((NPAD, 2 * F1), jnp.float32))

    srct2, adx2 = _tc2(acc1, b1.reshape(1, F1), W2,
                       att_src2.reshape(OUT_CH, 1), att_dst2.reshape(OUT_CH, 1))

    sc2 = _make_sc_pass(epad, OUT_CH)
    acc2 = sc2(src, dst, srct2, adx2,
               jnp.zeros((NPAD, 2 * OUT_CH), jnp.float32))

    out = _tc3(acc2, b2.reshape(1, OUT_CH))
    return out[:N]
